# baseline (device time: 9758 ns/iter reference)
import jax
import jax.numpy as jnp
from jax import lax
from jax.experimental import pallas as pl
from jax.experimental.pallas import tpu as pltpu

N_DEV = 8
BLK = 256


def kernel(x):
    m, n = x.shape
    n_blk = m // BLK

    def body(x_ref, out_ref, totals_ref, send_sems, recv_sems):
        my = lax.axis_index("i")

        barrier_sem = pltpu.get_barrier_semaphore()
        for k in range(N_DEV):

            @pl.when(my != k)
            def _():
                pl.semaphore_signal(
                    barrier_sem,
                    inc=1,
                    device_id=(k,),
                    device_id_type=pl.DeviceIdType.MESH,
                )

        total = jnp.sum(x_ref[...], axis=0, keepdims=True)
        for j in range(N_DEV):

            @pl.when(my == j)
            def _():
                totals_ref[pl.ds(j, 1), :] = total

        r = lax.broadcasted_iota(jnp.int32, (BLK, BLK), 0)
        c = lax.broadcasted_iota(jnp.int32, (BLK, BLK), 1)
        tri = jnp.where(r >= c, 1.0, 0.0).astype(jnp.bfloat16)
        local_carry = jnp.zeros((1, n), jnp.float32)
        for b in range(n_blk):
            xb = x_ref[pl.ds(b * BLK, BLK), :].astype(jnp.bfloat16)
            yb = jnp.dot(tri, xb, preferred_element_type=jnp.float32)
            out_ref[pl.ds(b * BLK, BLK), :] = yb + local_carry
            local_carry = local_carry + yb[BLK - 1 : BLK, :]

        pl.semaphore_wait(barrier_sem, N_DEV - 1)
        for j in range(N_DEV):

            @pl.when(my == j)
            def _():
                for k in range(N_DEV):
                    if k == j:
                        continue
                    rdma = pltpu.make_async_remote_copy(
                        src_ref=totals_ref.at[pl.ds(j, 1)],
                        dst_ref=totals_ref.at[pl.ds(j, 1)],
                        send_sem=send_sems.at[k],
                        recv_sem=recv_sems.at[j],
                        device_id=(k,),
                        device_id_type=pl.DeviceIdType.MESH,
                    )
                    rdma.start()

        for j in range(N_DEV):

            @pl.when(my != j)
            def _():
                recv = pltpu.make_async_remote_copy(
                    src_ref=totals_ref.at[pl.ds(j, 1)],
                    dst_ref=totals_ref.at[pl.ds(j, 1)],
                    send_sem=send_sems.at[j],
                    recv_sem=recv_sems.at[j],
                    device_id=(j,),
                    device_id_type=pl.DeviceIdType.MESH,
                )
                recv.wait_recv()

        row = lax.broadcasted_iota(jnp.int32, (N_DEV, n), 0)
        carry = jnp.sum(
            jnp.where(row < my, totals_ref[...], 0.0), axis=0, keepdims=True
        )
        out_ref[...] = out_ref[...] + carry

        for k in range(N_DEV):

            @pl.when(my != k)
            def _():
                sent = pltpu.make_async_remote_copy(
                    src_ref=totals_ref.at[pl.ds(0, 1)],
                    dst_ref=totals_ref.at[pl.ds(0, 1)],
                    send_sem=send_sems.at[k],
                    recv_sem=recv_sems.at[k],
                    device_id=(k,),
                    device_id_type=pl.DeviceIdType.MESH,
                )
                sent.wait_send()

    return pl.pallas_call(
        body,
        out_shape=jax.ShapeDtypeStruct((m, n), jnp.float32),
        in_specs=[pl.BlockSpec(memory_space=pltpu.VMEM)],
        out_specs=pl.BlockSpec(memory_space=pltpu.VMEM),
        scratch_shapes=[
            pltpu.VMEM((N_DEV, n), jnp.float32),
            pltpu.SemaphoreType.DMA((N_DEV,)),
            pltpu.SemaphoreType.DMA((N_DEV,)),
        ],
        compiler_params=pltpu.CompilerParams(collective_id=0),
    )(x)


# device time: 9706 ns/iter; 1.0054x vs baseline; 1.0054x over previous
import jax
import jax.numpy as jnp
from jax import lax
from jax.experimental import pallas as pl
from jax.experimental.pallas import tpu as pltpu

N_DEV = 8
BLK = 256


def kernel(x):
    m, n = x.shape
    n_blk = m // BLK

    def body(x_ref, out_ref, totals_ref, send_sems, recv_sems):
        my = lax.axis_index("i")

        barrier_sem = pltpu.get_barrier_semaphore()
        right = (my + 1) % N_DEV
        pl.semaphore_signal(
            barrier_sem,
            inc=1,
            device_id=(right,),
            device_id_type=pl.DeviceIdType.MESH,
        )

        total = jnp.sum(x_ref[...], axis=0, keepdims=True)
        for j in range(N_DEV):

            @pl.when(my == j)
            def _():
                totals_ref[pl.ds(j, 1), :] = total

        r = lax.broadcasted_iota(jnp.int32, (BLK, BLK), 0)
        c = lax.broadcasted_iota(jnp.int32, (BLK, BLK), 1)
        tri = jnp.where(r >= c, 1.0, 0.0).astype(jnp.bfloat16)
        local_carry = jnp.zeros((1, n), jnp.float32)
        for b in range(n_blk):
            xb = x_ref[pl.ds(b * BLK, BLK), :].astype(jnp.bfloat16)
            yb = jnp.dot(tri, xb, preferred_element_type=jnp.float32)
            out_ref[pl.ds(b * BLK, BLK), :] = yb + local_carry
            local_carry = local_carry + yb[BLK - 1 : BLK, :]

        pl.semaphore_wait(barrier_sem, 1)
        for j in range(N_DEV):

            @pl.when(my == j)
            def _():
                for k in range(N_DEV):
                    if k == j:
                        continue
                    rdma = pltpu.make_async_remote_copy(
                        src_ref=totals_ref.at[pl.ds(j, 1)],
                        dst_ref=totals_ref.at[pl.ds(j, 1)],
                        send_sem=send_sems.at[k],
                        recv_sem=recv_sems.at[j],
                        device_id=(k,),
                        device_id_type=pl.DeviceIdType.MESH,
                    )
                    rdma.start()

        for j in range(N_DEV):

            @pl.when(my != j)
            def _():
                recv = pltpu.make_async_remote_copy(
                    src_ref=totals_ref.at[pl.ds(j, 1)],
                    dst_ref=totals_ref.at[pl.ds(j, 1)],
                    send_sem=send_sems.at[j],
                    recv_sem=recv_sems.at[j],
                    device_id=(j,),
                    device_id_type=pl.DeviceIdType.MESH,
                )
                recv.wait_recv()

        row = lax.broadcasted_iota(jnp.int32, (N_DEV, n), 0)
        carry = jnp.sum(
            jnp.where(row < my, totals_ref[...], 0.0), axis=0, keepdims=True
        )
        out_ref[...] = out_ref[...] + carry

        for k in range(N_DEV):

            @pl.when(my != k)
            def _():
                sent = pltpu.make_async_remote_copy(
                    src_ref=totals_ref.at[pl.ds(0, 1)],
                    dst_ref=totals_ref.at[pl.ds(0, 1)],
                    send_sem=send_sems.at[k],
                    recv_sem=recv_sems.at[k],
                    device_id=(k,),
                    device_id_type=pl.DeviceIdType.MESH,
                )
                sent.wait_send()

    return pl.pallas_call(
        body,
        out_shape=jax.ShapeDtypeStruct((m, n), jnp.float32),
        in_specs=[pl.BlockSpec(memory_space=pltpu.VMEM)],
        out_specs=pl.BlockSpec(memory_space=pltpu.VMEM),
        scratch_shapes=[
            pltpu.VMEM((N_DEV, n), jnp.float32),
            pltpu.SemaphoreType.DMA((N_DEV,)),
            pltpu.SemaphoreType.DMA((N_DEV,)),
        ],
        compiler_params=pltpu.CompilerParams(collective_id=0),
    )(x)


# device time: 9486 ns/iter; 1.0287x vs baseline; 1.0232x over previous
import jax
import jax.numpy as jnp
from jax import lax
from jax.experimental import pallas as pl
from jax.experimental.pallas import tpu as pltpu

N_DEV = 8
BLK = 256


def kernel(x):
    m, n = x.shape
    n_blk = m // BLK

    def body(x_ref, out_ref, totals_ref, send_sems, recv_sems):
        my = lax.axis_index("i")

        barrier_sem = pltpu.get_barrier_semaphore()
        for k in range(N_DEV):

            @pl.when(my != k)
            def _():
                pl.semaphore_signal(
                    barrier_sem,
                    inc=1,
                    device_id=(k,),
                    device_id_type=pl.DeviceIdType.MESH,
                )

        total = jnp.sum(x_ref[...], axis=0, keepdims=True)
        for j in range(N_DEV):

            @pl.when(my == j)
            def _():
                totals_ref[pl.ds(j, 1), :] = total

        r = lax.broadcasted_iota(jnp.int32, (BLK, BLK), 0)
        c = lax.broadcasted_iota(jnp.int32, (BLK, BLK), 1)
        tri = jnp.where(r >= c, 1.0, 0.0).astype(jnp.bfloat16)

        def cumsum_block(b, local_carry):
            xb = x_ref[pl.ds(b * BLK, BLK), :].astype(jnp.bfloat16)
            yb = jnp.dot(tri, xb, preferred_element_type=jnp.float32)
            out_ref[pl.ds(b * BLK, BLK), :] = yb + local_carry
            return local_carry + yb[BLK - 1 : BLK, :]

        local_carry = jnp.zeros((1, n), jnp.float32)
        for b in range(n_blk // 2):
            local_carry = cumsum_block(b, local_carry)

        pl.semaphore_wait(barrier_sem, N_DEV - 1)
        for j in range(N_DEV):

            @pl.when(my == j)
            def _():
                for k in list(range(j + 1, N_DEV)) + list(range(j)):
                    rdma = pltpu.make_async_remote_copy(
                        src_ref=totals_ref.at[pl.ds(j, 1)],
                        dst_ref=totals_ref.at[pl.ds(j, 1)],
                        send_sem=send_sems.at[k],
                        recv_sem=recv_sems.at[j],
                        device_id=(k,),
                        device_id_type=pl.DeviceIdType.MESH,
                    )
                    rdma.start()

        for b in range(n_blk // 2, n_blk):
            local_carry = cumsum_block(b, local_carry)

        def wait_recv_from(j):
            recv = pltpu.make_async_remote_copy(
                src_ref=totals_ref.at[pl.ds(j, 1)],
                dst_ref=totals_ref.at[pl.ds(j, 1)],
                send_sem=send_sems.at[j],
                recv_sem=recv_sems.at[j],
                device_id=(j,),
                device_id_type=pl.DeviceIdType.MESH,
            )
            recv.wait_recv()

        for j in range(N_DEV):

            @pl.when(j < my)
            def _():
                wait_recv_from(j)

        row = lax.broadcasted_iota(jnp.int32, (N_DEV, n), 0)
        carry = jnp.sum(
            jnp.where(row < my, totals_ref[...], 0.0), axis=0, keepdims=True
        )
        out_ref[...] = out_ref[...] + carry

        for j in range(N_DEV):

            @pl.when(j > my)
            def _():
                wait_recv_from(j)

        for k in range(N_DEV):

            @pl.when(my != k)
            def _():
                sent = pltpu.make_async_remote_copy(
                    src_ref=totals_ref.at[pl.ds(0, 1)],
                    dst_ref=totals_ref.at[pl.ds(0, 1)],
                    send_sem=send_sems.at[k],
                    recv_sem=recv_sems.at[k],
                    device_id=(k,),
                    device_id_type=pl.DeviceIdType.MESH,
                )
                sent.wait_send()

    return pl.pallas_call(
        body,
        out_shape=jax.ShapeDtypeStruct((m, n), jnp.float32),
        in_specs=[pl.BlockSpec(memory_space=pltpu.VMEM)],
        out_specs=pl.BlockSpec(memory_space=pltpu.VMEM),
        scratch_shapes=[
            pltpu.VMEM((N_DEV, n), jnp.float32),
            pltpu.SemaphoreType.DMA((N_DEV,)),
            pltpu.SemaphoreType.DMA((N_DEV,)),
        ],
        compiler_params=pltpu.CompilerParams(collective_id=0),
    )(x)
